# baseline (speedup 1.0000x reference)
import functools

import jax
import jax.numpy as jnp
from jax import lax
from jax.experimental import pallas as pl
from jax.experimental.pallas import tpu as pltpu
from jax.experimental.pallas import tpu_sc as plsc

_D = 32
_CH = 128
_NW = 32
_NB = 4
_NT = 2


@functools.partial(jax.jit, static_argnums=(2, 3, 4))
def _sc_gather_fast(ids3, table, nb, p_dim, ncol):
    mesh = plsc.VectorSubcoreMesh(core_axis_name="c", subcore_axis_name="s")
    ng = nb // _NB

    @functools.partial(
        pl.kernel,
        mesh=mesh,
        out_type=jax.ShapeDtypeStruct((p_dim, 4, ncol, 8, _CH), jnp.float32),
        scratch_types=(
            [pltpu.VMEM((nb, _CH), jnp.int32)]
            + [pltpu.VMEM((_CH, _D), jnp.float32) for _ in range(_NB)]
            + [pltpu.VMEM((4, 8, _CH), jnp.float32) for _ in range(_NT)]
            + [pltpu.SemaphoreType.DMA for _ in range(_NB + _NT)]
        ),
        compiler_params=pltpu.CompilerParams(use_tc_tiling_on_sc=False, needs_layout_passes=False),
    )
    def body(ids_hbm, table_hbm, out_hbm, idx_v, *rest):
        gbufs = rest[:_NB]
        tbufs = rest[_NB:_NB + _NT]
        gsems = rest[_NB + _NT:_NB + _NT + _NB]
        ssems = rest[_NB + _NT + _NB:]
        wid = lax.axis_index("s") * 2 + lax.axis_index("c")
        pltpu.sync_copy(ids_hbm.at[wid], idx_v)
        base_iota = lax.iota(jnp.int32, 16)
        ti_vs = [base_iota + (chunk * 16) for chunk in range(8)]

        def fire_gather(jj, b):
            pltpu.async_copy(table_hbm.at[idx_v.at[jj]], gbufs[b], gsems[b])

        for b in range(_NB):
            fire_gather(b, b)

        def store_wait(bg2, tb):
            pltpu.make_async_copy(
                tbufs[tb],
                out_hbm.at[bg2 // ncol, :, lax.rem(bg2, ncol)],
                ssems[tb],
            ).wait()

        def step(g, carry):
            for b in range(_NB):
                jj = g * _NB + b
                tb = b % _NT
                bg = wid * nb + jj
                pltpu.make_async_copy(
                    table_hbm.at[pl.ds(0, _CH)], gbufs[b], gsems[b]
                ).wait()

                @pl.when(jj >= _NT)
                def _():
                    store_wait(bg - _NT, tb)

                gbuf = gbufs[b]
                tbuf = tbufs[tb]
                def tstep(f0u, c2, gbuf=gbuf, tbuf=tbuf):
                    for df in range(8):
                        f_idx = lax.rem(base_iota + (f0u * 8 + df), _D)
                        fr_idx = f_idx // 8
                        fi_idx = lax.rem(f_idx, 8)
                        for chunk in range(8):
                            vec = plsc.load_gather(gbuf, [ti_vs[chunk], f_idx])
                            plsc.store_scatter(
                                tbuf, [fr_idx, fi_idx, ti_vs[chunk]], vec
                            )
                    return c2

                lax.fori_loop(0, _D // 8, tstep, 0)

                @pl.when(jj + _NB < nb)
                def _():
                    fire_gather(jj + _NB, b)

                pltpu.async_copy(
                    tbuf,
                    out_hbm.at[bg // ncol, :, lax.rem(bg, ncol)],
                    ssems[tb],
                )
            return carry

        lax.fori_loop(0, ng, step, 0)
        for e in range(_NT):
            store_wait(wid * nb + nb - _NT + e, (nb - _NT + e) % _NT)

    return body(ids3, table)


def _fast(token_ids, embedding_matrix):
    t_dim, p_dim = token_ids.shape
    ncol = t_dim // _CH
    blocks = p_dim * ncol
    nb = blocks // _NW
    tpad = jnp.pad(embedding_matrix, ((0, 0), (0, 128 - _D)))
    t4 = tpad.reshape(embedding_matrix.shape[0] * 4, _D)
    ids_pm = (token_ids.T.astype(jnp.int32) * 4).reshape(-1)
    ids3 = ids_pm.reshape(_NW, nb, _CH)
    out5 = _sc_gather_fast(ids3, t4, nb, p_dim, ncol)
    return out5.transpose(2, 4, 0, 1, 3).reshape(t_dim, p_dim, _D)


_G = 10
_GR = _G * _CH


@functools.partial(jax.jit, static_argnums=(2, 3))
def _sc_gather_generic(ids3, table, nw, k):
    npair = k // (2 * _G)
    rows_w = k * _CH
    mesh = plsc.VectorSubcoreMesh(core_axis_name="c", subcore_axis_name="s")

    @functools.partial(
        pl.kernel,
        mesh=mesh,
        out_type=jax.ShapeDtypeStruct((nw * rows_w, _D), jnp.float32),
        scratch_types=[
            pltpu.VMEM((k, _CH), jnp.int32),
            pltpu.VMEM((_GR, _D), jnp.float32),
            pltpu.VMEM((_GR, _D), jnp.float32),
            pltpu.SemaphoreType.DMA,
            pltpu.SemaphoreType.DMA,
            pltpu.SemaphoreType.DMA,
        ],
        compiler_params=pltpu.CompilerParams(use_tc_tiling_on_sc=False, needs_layout_passes=False),
    )
    def body(ids_hbm, table_hbm, out_hbm, idx_v, buf_a, buf_b, gsem, ssem_a, ssem_b):
        wid = lax.axis_index("s") * 2 + lax.axis_index("c")
        pltpu.sync_copy(ids_hbm.at[wid], idx_v)

        def fill(buf, g):
            handles = []
            for t in range(_G):
                j = g * _G + t
                handles.append(
                    pltpu.async_copy(
                        table_hbm.at[idx_v.at[j]],
                        buf.at[pl.ds(t * _CH, _CH)],
                        gsem,
                    )
                )
            for h in handles:
                h.wait()

        def store_start(buf, g, sem):
            pltpu.async_copy(buf, out_hbm.at[pl.ds(wid * rows_w + g * _GR, _GR)], sem)

        def store_wait(buf, g, sem):
            pltpu.make_async_copy(
                buf, out_hbm.at[pl.ds(wid * rows_w + g * _GR, _GR)], sem
            ).wait()

        def pair(p, carry):
            g0 = 2 * p
            g1 = g0 + 1

            @pl.when(p > 0)
            def _():
                store_wait(buf_a, g0 - 2, ssem_a)

            fill(buf_a, g0)
            store_start(buf_a, g0, ssem_a)

            @pl.when(p > 0)
            def _():
                store_wait(buf_b, g1 - 2, ssem_b)

            fill(buf_b, g1)
            store_start(buf_b, g1, ssem_b)
            return carry

        lax.fori_loop(0, npair, pair, 0)
        store_wait(buf_a, 2 * npair - 2, ssem_a)
        store_wait(buf_b, 2 * npair - 1, ssem_b)

    return body(ids3, table)


def _generic(token_ids, embedding_matrix):
    d = embedding_matrix.shape[1]
    ids = token_ids.reshape(-1).astype(jnp.int32)
    b = ids.shape[0]
    chunk = _NW * _CH * 2 * _G
    k_pairs = -(-b // chunk)
    pad = k_pairs * chunk - b
    if pad:
        ids = jnp.concatenate([ids, jnp.zeros((pad,), jnp.int32)])
    k = k_pairs * 2 * _G
    ids3 = ids.reshape(_NW, k, _CH)
    out = _sc_gather_generic(ids3, embedding_matrix, _NW, k)
    if pad:
        out = out[:b]
    return out.reshape(token_ids.shape + (d,))


def kernel(token_ids, embedding_matrix):
    if (
        token_ids.ndim == 2
        and embedding_matrix.shape[1] == _D
        and token_ids.shape[0] % _CH == 0
        and (token_ids.shape[1] * (token_ids.shape[0] // _CH)) % (_NW * _NB) == 0
    ):
        return _fast(token_ids, embedding_matrix)
    return _generic(token_ids, embedding_matrix)


# speedup vs baseline: 1.1186x; 1.1186x over previous
import functools

import jax
import jax.numpy as jnp
from jax import lax
from jax.experimental import pallas as pl
from jax.experimental.pallas import tpu as pltpu
from jax.experimental.pallas import tpu_sc as plsc

_D = 32
_CH = 128
_NW = 32
_NB = 4
_NT = 2


@functools.partial(jax.jit, static_argnums=(2, 3, 4))
def _sc_gather_fast(ids3, table, nb, p_dim, ncol):
    mesh = plsc.VectorSubcoreMesh(core_axis_name="c", subcore_axis_name="s")
    ng = nb // _NB

    @functools.partial(
        pl.kernel,
        mesh=mesh,
        out_type=jax.ShapeDtypeStruct((p_dim, 4, ncol, 8, _CH), jnp.float32),
        scratch_types=(
            [pltpu.VMEM((nb, _CH), jnp.int32)]
            + [pltpu.VMEM((_CH, _D), jnp.float32) for _ in range(_NB)]
            + [pltpu.VMEM((4, 8, _CH), jnp.float32) for _ in range(_NT)]
            + [pltpu.SemaphoreType.DMA for _ in range(_NB + _NT)]
        ),
        compiler_params=pltpu.CompilerParams(use_tc_tiling_on_sc=False, needs_layout_passes=False),
    )
    def body(ids_hbm, table_hbm, out_hbm, idx_v, *rest):
        gbufs = rest[:_NB]
        tbufs = rest[_NB:_NB + _NT]
        gsems = rest[_NB + _NT:_NB + _NT + _NB]
        ssems = rest[_NB + _NT + _NB:]
        wid = lax.axis_index("s") * 2 + lax.axis_index("c")
        pltpu.sync_copy(ids_hbm.at[wid], idx_v)
        base_iota = lax.iota(jnp.int32, 16)
        ti_vs = [base_iota + (chunk * 16) for chunk in range(8)]

        def fire_gather(jj, b):
            pltpu.async_copy(table_hbm.at[idx_v.at[jj]], gbufs[b], gsems[b])

        for b in range(_NB):
            fire_gather(b, b)

        def store_wait(bg2, tb):
            pltpu.make_async_copy(
                tbufs[tb],
                out_hbm.at[bg2 // ncol, :, lax.rem(bg2, ncol)],
                ssems[tb],
            ).wait()

        def step(g, carry):
            for b in range(_NB):
                jj = g * _NB + b
                tb = b % _NT
                bg = wid * nb + jj
                pltpu.make_async_copy(
                    table_hbm.at[pl.ds(0, _CH)], gbufs[b], gsems[b]
                ).wait()

                @pl.when(jj >= _NT)
                def _():
                    store_wait(bg - _NT, tb)

                gbuf = gbufs[b]
                tbuf = tbufs[tb]
                def tstep(f0u, c2, gbuf=gbuf, tbuf=tbuf):
                    for df in range(4):
                        f_idx = lax.rem(base_iota + (f0u * 4 + df), _D)
                        fr_idx = f_idx // 8
                        fi_idx = lax.rem(f_idx, 8)
                        for chunk in range(8):
                            vec = plsc.load_gather(gbuf, [ti_vs[chunk], f_idx])
                            plsc.store_scatter(
                                tbuf, [fr_idx, fi_idx, ti_vs[chunk]], vec
                            )
                    return c2

                lax.fori_loop(0, _D // 4, tstep, 0)

                @pl.when(jj + _NB < nb)
                def _():
                    fire_gather(jj + _NB, b)

                pltpu.async_copy(
                    tbuf,
                    out_hbm.at[bg // ncol, :, lax.rem(bg, ncol)],
                    ssems[tb],
                )
            return carry

        lax.fori_loop(0, ng, step, 0)
        for e in range(_NT):
            store_wait(wid * nb + nb - _NT + e, (nb - _NT + e) % _NT)

    return body(ids3, table)


def _fast(token_ids, embedding_matrix):
    t_dim, p_dim = token_ids.shape
    ncol = t_dim // _CH
    blocks = p_dim * ncol
    nb = blocks // _NW
    tpad = jnp.pad(embedding_matrix, ((0, 0), (0, 128 - _D)))
    t4 = tpad.reshape(embedding_matrix.shape[0] * 4, _D)
    ids_pm = (token_ids.T.astype(jnp.int32) * 4).reshape(-1)
    ids3 = ids_pm.reshape(_NW, nb, _CH)
    out5 = _sc_gather_fast(ids3, t4, nb, p_dim, ncol)
    return out5.transpose(2, 4, 0, 1, 3).reshape(t_dim, p_dim, _D)


_G = 10
_GR = _G * _CH


@functools.partial(jax.jit, static_argnums=(2, 3))
def _sc_gather_generic(ids3, table, nw, k):
    npair = k // (2 * _G)
    rows_w = k * _CH
    mesh = plsc.VectorSubcoreMesh(core_axis_name="c", subcore_axis_name="s")

    @functools.partial(
        pl.kernel,
        mesh=mesh,
        out_type=jax.ShapeDtypeStruct((nw * rows_w, _D), jnp.float32),
        scratch_types=[
            pltpu.VMEM((k, _CH), jnp.int32),
            pltpu.VMEM((_GR, _D), jnp.float32),
            pltpu.VMEM((_GR, _D), jnp.float32),
            pltpu.SemaphoreType.DMA,
            pltpu.SemaphoreType.DMA,
            pltpu.SemaphoreType.DMA,
        ],
        compiler_params=pltpu.CompilerParams(use_tc_tiling_on_sc=False, needs_layout_passes=False),
    )
    def body(ids_hbm, table_hbm, out_hbm, idx_v, buf_a, buf_b, gsem, ssem_a, ssem_b):
        wid = lax.axis_index("s") * 2 + lax.axis_index("c")
        pltpu.sync_copy(ids_hbm.at[wid], idx_v)

        def fill(buf, g):
            handles = []
            for t in range(_G):
                j = g * _G + t
                handles.append(
                    pltpu.async_copy(
                        table_hbm.at[idx_v.at[j]],
                        buf.at[pl.ds(t * _CH, _CH)],
                        gsem,
                    )
                )
            for h in handles:
                h.wait()

        def store_start(buf, g, sem):
            pltpu.async_copy(buf, out_hbm.at[pl.ds(wid * rows_w + g * _GR, _GR)], sem)

        def store_wait(buf, g, sem):
            pltpu.make_async_copy(
                buf, out_hbm.at[pl.ds(wid * rows_w + g * _GR, _GR)], sem
            ).wait()

        def pair(p, carry):
            g0 = 2 * p
            g1 = g0 + 1

            @pl.when(p > 0)
            def _():
                store_wait(buf_a, g0 - 2, ssem_a)

            fill(buf_a, g0)
            store_start(buf_a, g0, ssem_a)

            @pl.when(p > 0)
            def _():
                store_wait(buf_b, g1 - 2, ssem_b)

            fill(buf_b, g1)
            store_start(buf_b, g1, ssem_b)
            return carry

        lax.fori_loop(0, npair, pair, 0)
        store_wait(buf_a, 2 * npair - 2, ssem_a)
        store_wait(buf_b, 2 * npair - 1, ssem_b)

    return body(ids3, table)


def _generic(token_ids, embedding_matrix):
    d = embedding_matrix.shape[1]
    ids = token_ids.reshape(-1).astype(jnp.int32)
    b = ids.shape[0]
    chunk = _NW * _CH * 2 * _G
    k_pairs = -(-b // chunk)
    pad = k_pairs * chunk - b
    if pad:
        ids = jnp.concatenate([ids, jnp.zeros((pad,), jnp.int32)])
    k = k_pairs * 2 * _G
    ids3 = ids.reshape(_NW, k, _CH)
    out = _sc_gather_generic(ids3, embedding_matrix, _NW, k)
    if pad:
        out = out[:b]
    return out.reshape(token_ids.shape + (d,))


def kernel(token_ids, embedding_matrix):
    if (
        token_ids.ndim == 2
        and embedding_matrix.shape[1] == _D
        and token_ids.shape[0] % _CH == 0
        and (token_ids.shape[1] * (token_ids.shape[0] // _CH)) % (_NW * _NB) == 0
    ):
        return _fast(token_ids, embedding_matrix)
    return _generic(token_ids, embedding_matrix)
